# Initial kernel scaffold; baseline (speedup 1.0000x reference)
import jax
import jax.numpy as jnp
from jax.experimental import pallas as pl


def kernel(x, ei0, w0_lap, ei1, w1_lap, cW0, cb0, g0, be0, cW1, cb1, g1, be1, Wh, bh, Wo, bo):
    raise NotImplementedError("write your pallas kernel here")


# speedup vs baseline: 210.9028x; 210.9028x over previous
import functools

import jax
import jax.numpy as jnp
from jax import lax
from jax.experimental import pallas as pl
from jax.experimental.pallas import tpu as pltpu
from jax.experimental.pallas import tpu_sc as plsc

_P0 = 49152
_P1 = 12288
_DEG = 8
_B = 4
_LANES = 16


def _make_sc_lap(P, D, fused):
    ncores, nsub = 2, 16
    nw = ncores * nsub
    nodes_w = P // nw
    e_w = nodes_w * _DEG
    g_w = e_w // 128
    ec = 32768 // D
    cn = ec // _DEG
    nch = nodes_w // cn
    gpc = ec // 128
    nv = D // _LANES

    mesh = plsc.VectorSubcoreMesh(core_axis_name="c", subcore_axis_name="s",
                                  num_cores=ncores, num_subcores=nsub)
    out_type = jax.ShapeDtypeStruct((P, D), jnp.float32)
    scratch = [
        pltpu.VMEM((g_w, 128), jnp.int32),
        pltpu.VMEM((e_w + 8,), jnp.float32),
        pltpu.VMEM((2, ec, D), jnp.float32),
        pltpu.VMEM((cn, D), jnp.float32),
        pltpu.VMEM((cn, D), jnp.float32),
        pltpu.SemaphoreType.DMA,
        pltpu.SemaphoreType.DMA,
    ]

    def body(*refs):
        if fused:
            (table, src, w, x0h, out, idx_v, w_v, rows_v, out_v, x0_v,
             sem0, sem1) = refs
        else:
            (table, src, w, out, idx_v, w_v, rows_v, out_v, x0_v,
             sem0, sem1) = refs
            x0h = None
        wid = lax.axis_index("s") * ncores + lax.axis_index("c")
        nbase = wid * nodes_w
        pltpu.sync_copy(src.at[pl.ds(wid * g_w, g_w)], idx_v)
        pltpu.sync_copy(w.at[pl.ds(wid * e_w, e_w)], w_v.at[pl.ds(0, e_w)])
        sems = [sem0, sem1]

        def issue(c, slot):
            descs = []
            for g in range(gpc):
                descs.append(pltpu.async_copy(
                    table.at[idx_v.at[c * gpc + g]],
                    rows_v.at[slot, pl.ds(g * 128, 128)],
                    sems[slot]))
            return descs

        pending = {0: issue(0, 0)}
        for c in range(nch):
            slot = c % 2
            if c + 1 < nch:
                pending[c + 1] = issue(c + 1, (c + 1) % 2)
            for dsc in pending.pop(c):
                dsc.wait()
            if fused:
                pltpu.sync_copy(x0h.at[pl.ds(nbase + c * cn, cn)], x0_v)

            def node_body(n, carry, slot=slot, c=c):
                e0 = n * _DEG
                w16 = w_v[pl.ds(c * ec + e0, _LANES)]
                accs = [jnp.zeros((_LANES,), jnp.float32)
                        for _ in range(nv)]
                for d in range(_DEG):
                    wv = w16[d]
                    for j in range(nv):
                        row = rows_v[slot, e0 + d, pl.ds(j * _LANES, _LANES)]
                        accs[j] = accs[j] + row * wv
                for j in range(nv):
                    if fused:
                        res = 2.0 * accs[j] - x0_v[n, pl.ds(j * _LANES,
                                                            _LANES)]
                    else:
                        res = accs[j]
                    out_v[n, pl.ds(j * _LANES, _LANES)] = res
                return carry

            lax.fori_loop(0, cn, node_body, 0)
            pltpu.sync_copy(out_v, out.at[pl.ds(nbase + c * cn, cn)])

    return pl.kernel(body, out_type=out_type, mesh=mesh,
                     scratch_types=scratch,
                     compiler_params=pltpu.CompilerParams(
                         use_tc_tiling_on_sc=False))


def _combine0(x0, x1, x2, wcat, b, g, be):
    tp = 512

    def body(x0r, x1r, x2r, wr, br, gr, ber, outr):
        xx = jnp.concatenate([x0r[...], x1r[...], x2r[...]], axis=-1)
        y = jnp.dot(xx, wr[...], preferred_element_type=jnp.float32) + br[...]
        m = jnp.mean(y, axis=-1, keepdims=True)
        v = jnp.mean((y - m) ** 2, axis=-1, keepdims=True)
        y = (y - m) * lax.rsqrt(v + 1e-5) * gr[...] + ber[...]
        y = jnp.maximum(y, 0.0)
        ys = y.reshape(tp // 4, 16, 32)
        pooled = (ys[:, 0:4] + ys[:, 4:8] + ys[:, 8:12] + ys[:, 12:16]) * 0.25
        outr[...] = pooled.reshape(tp, 32)

    grid = _P0 // tp
    return pl.pallas_call(
        body,
        grid=(grid,),
        in_specs=[
            pl.BlockSpec((tp * _B, 16), lambda i: (i, 0)),
            pl.BlockSpec((tp * _B, 16), lambda i: (i, 0)),
            pl.BlockSpec((tp * _B, 16), lambda i: (i, 0)),
            pl.BlockSpec((48, 32), lambda i: (0, 0)),
            pl.BlockSpec((1, 32), lambda i: (0, 0)),
            pl.BlockSpec((1, 32), lambda i: (0, 0)),
            pl.BlockSpec((1, 32), lambda i: (0, 0)),
        ],
        out_specs=pl.BlockSpec((tp, 32), lambda i: (i, 0)),
        out_shape=jax.ShapeDtypeStruct((_P1 * _B, 32), jnp.float32),
    )(x0, x1, x2, wcat, b, g, be)


def _combine1(x0, x1, x2, wcat, b, g, be):
    tp = 512

    def body(x0r, x1r, x2r, wr, br, gr, ber, outr):
        xx = jnp.concatenate([x0r[...], x1r[...], x2r[...]], axis=-1)
        y = jnp.dot(xx, wr[...], preferred_element_type=jnp.float32) + br[...]
        m = jnp.mean(y, axis=-1, keepdims=True)
        v = jnp.mean((y - m) ** 2, axis=-1, keepdims=True)
        y = (y - m) * lax.rsqrt(v + 1e-5) * gr[...] + ber[...]
        y = jnp.maximum(y, 0.0)
        part = jnp.sum(y.reshape(tp, _B, 32), axis=0)

        @pl.when(pl.program_id(0) == 0)
        def _():
            outr[...] = jnp.zeros_like(outr)

        outr[...] += part

    grid = _P1 // tp
    return pl.pallas_call(
        body,
        grid=(grid,),
        in_specs=[
            pl.BlockSpec((tp * _B, 32), lambda i: (i, 0)),
            pl.BlockSpec((tp * _B, 32), lambda i: (i, 0)),
            pl.BlockSpec((tp * _B, 32), lambda i: (i, 0)),
            pl.BlockSpec((96, 32), lambda i: (0, 0)),
            pl.BlockSpec((1, 32), lambda i: (0, 0)),
            pl.BlockSpec((1, 32), lambda i: (0, 0)),
            pl.BlockSpec((1, 32), lambda i: (0, 0)),
        ],
        out_specs=pl.BlockSpec((_B, 32), lambda i: (0, 0)),
        out_shape=jax.ShapeDtypeStruct((_B, 32), jnp.float32),
    )(x0, x1, x2, wcat, b, g, be)


def _mlp_head(zsum, wh, bh, wo, bo):
    def body(zr, whr, bhr, wor, bor, outr):
        z = zr[...] * (1.0 / _P1)
        h = jnp.maximum(
            jnp.dot(z, whr[...], preferred_element_type=jnp.float32)
            + bhr[...], 0.0)
        outr[...] = (jnp.dot(h, wor[...], preferred_element_type=jnp.float32)
                     + bor[...])

    return pl.pallas_call(
        body,
        out_shape=jax.ShapeDtypeStruct((_B, 128), jnp.float32),
    )(zsum, wh, bh, wo, bo)


def kernel(x, ei0, w0_lap, ei1, w1_lap, cW0, cb0, g0, be0,
           cW1, cb1, g1, be1, Wh, bh, Wo, bo):
    lap0 = _make_sc_lap(_P0, _B * 16, fused=False)
    lap0f = _make_sc_lap(_P0, _B * 16, fused=True)
    lap1 = _make_sc_lap(_P1, _B * 32, fused=False)
    lap1f = _make_sc_lap(_P1, _B * 32, fused=True)

    xt = jnp.transpose(x, (1, 0, 2)).reshape(_P0, _B * 16)
    src0 = ei0[0].reshape(-1, 128)
    src1 = ei1[0].reshape(-1, 128)

    x1t = lap0(xt, src0, w0_lap)
    x2t = lap0f(x1t, src0, w0_lap, xt)

    wcat0 = jnp.concatenate([cW0[0], cW0[1], cW0[2]], axis=0)
    h0 = _combine0(xt.reshape(_P0 * _B, 16),
                   x1t.reshape(_P0 * _B, 16),
                   x2t.reshape(_P0 * _B, 16),
                   wcat0, cb0.reshape(1, 32), g0.reshape(1, 32),
                   be0.reshape(1, 32))

    h0t = h0.reshape(_P1, _B * 32)
    h1t = lap1(h0t, src1, w1_lap)
    h2t = lap1f(h1t, src1, w1_lap, h0t)

    wcat1 = jnp.concatenate([cW1[0], cW1[1], cW1[2]], axis=0)
    zsum = _combine1(h0, h1t.reshape(_P1 * _B, 32),
                     h2t.reshape(_P1 * _B, 32),
                     wcat1, cb1.reshape(1, 32), g1.reshape(1, 32),
                     be1.reshape(1, 32))

    return _mlp_head(zsum, Wh, bh.reshape(1, 256), Wo, bo.reshape(1, 128))


# speedup vs baseline: 237.3507x; 1.1254x over previous
import functools

import jax
import jax.numpy as jnp
from jax import lax
from jax.experimental import pallas as pl
from jax.experimental.pallas import tpu as pltpu
from jax.experimental.pallas import tpu_sc as plsc

_P0 = 49152
_P1 = 12288
_DEG = 8
_B = 4
_LANES = 16


def _make_sc_lap(P, D):
    ncores, nsub = 2, 16
    nw = ncores * nsub
    nodes_w = P // nw
    e_w = nodes_w * _DEG
    g_w = e_w // 128
    ec = 32768 // D
    cn = ec // _DEG
    nch = nodes_w // cn
    gpc = ec // 128
    nv = D // _LANES

    mesh = plsc.VectorSubcoreMesh(core_axis_name="c", subcore_axis_name="s",
                                  num_cores=ncores, num_subcores=nsub)
    out_type = jax.ShapeDtypeStruct((P, D), jnp.float32)
    scratch = [
        pltpu.VMEM((g_w, 128), jnp.int32),
        pltpu.VMEM((e_w + 8,), jnp.float32),
        pltpu.VMEM((2, ec, D), jnp.float32),
        pltpu.VMEM((cn, D), jnp.float32),
        pltpu.SemaphoreType.DMA,
        pltpu.SemaphoreType.DMA,
    ]

    def body(table, src, w, out, idx_v, w_v, rows_v, out_v, sem0, sem1):
        wid = lax.axis_index("s") * ncores + lax.axis_index("c")
        nbase = wid * nodes_w
        pltpu.sync_copy(src.at[pl.ds(wid * g_w, g_w)], idx_v)
        pltpu.sync_copy(w.at[pl.ds(wid * e_w, e_w)], w_v.at[pl.ds(0, e_w)])
        sems = [sem0, sem1]

        def issue(c, slot):
            descs = []
            for g in range(gpc):
                descs.append(pltpu.async_copy(
                    table.at[idx_v.at[c * gpc + g]],
                    rows_v.at[slot, pl.ds(g * 128, 128)],
                    sems[slot]))
            return descs

        pending = {0: issue(0, 0)}
        for c in range(nch):
            slot = c % 2
            if c + 1 < nch:
                pending[c + 1] = issue(c + 1, (c + 1) % 2)
            for dsc in pending.pop(c):
                dsc.wait()

            def node_body(n, carry, slot=slot, c=c):
                e0 = n * _DEG
                w16 = w_v[pl.ds(c * ec + e0, _LANES)]
                accs = [jnp.zeros((_LANES,), jnp.float32)
                        for _ in range(nv)]
                for d in range(_DEG):
                    wv = w16[d]
                    for j in range(nv):
                        row = rows_v[slot, e0 + d, pl.ds(j * _LANES, _LANES)]
                        accs[j] = accs[j] + row * wv
                for j in range(nv):
                    out_v[n, pl.ds(j * _LANES, _LANES)] = accs[j]
                return carry

            lax.fori_loop(0, cn, node_body, 0)
            pltpu.sync_copy(out_v, out.at[pl.ds(nbase + c * cn, cn)])

    return pl.kernel(body, out_type=out_type, mesh=mesh,
                     scratch_types=scratch,
                     compiler_params=pltpu.CompilerParams(
                         use_tc_tiling_on_sc=False))


def _ln_relu(y, g, be):
    m = jnp.mean(y, axis=-1, keepdims=True)
    v = jnp.mean((y - m) ** 2, axis=-1, keepdims=True)
    return jnp.maximum((y - m) * lax.rsqrt(v + 1e-5) * g + be, 0.0)


def _combine0(x0, x1, x2, w0, w1, w2, b, g, be):
    tp = 512

    def body(x0r, x1r, x2r, w0r, w1r, w2r, br, gr, ber, outr):
        y = (jnp.dot(x0r[...], w0r[...], preferred_element_type=jnp.float32)
             + jnp.dot(x1r[...], w1r[...], preferred_element_type=jnp.float32)
             + jnp.dot(x2r[...], w2r[...], preferred_element_type=jnp.float32)
             + br[...])
        y = _ln_relu(y, gr[...], ber[...])
        ys = y.reshape(tp // 4, 16, 32)
        pooled = (ys[:, 0:4] + ys[:, 4:8] + ys[:, 8:12] + ys[:, 12:16]) * 0.25
        outr[...] = pooled.reshape(tp, 32)

    grid = _P0 // tp
    return pl.pallas_call(
        body,
        grid=(grid,),
        in_specs=[
            pl.BlockSpec((tp * _B, 16), lambda i: (i, 0)),
            pl.BlockSpec((tp * _B, 16), lambda i: (i, 0)),
            pl.BlockSpec((tp * _B, 16), lambda i: (i, 0)),
            pl.BlockSpec((16, 32), lambda i: (0, 0)),
            pl.BlockSpec((16, 32), lambda i: (0, 0)),
            pl.BlockSpec((16, 32), lambda i: (0, 0)),
            pl.BlockSpec((1, 32), lambda i: (0, 0)),
            pl.BlockSpec((1, 32), lambda i: (0, 0)),
            pl.BlockSpec((1, 32), lambda i: (0, 0)),
        ],
        out_specs=pl.BlockSpec((tp, 32), lambda i: (i, 0)),
        out_shape=jax.ShapeDtypeStruct((_P1 * _B, 32), jnp.float32),
    )(x0, x1, x2, w0, w1, w2, b, g, be)


def _combine1_head(x0, x1, x2, w0, w1, w2, b, g, be, wh, bh, wo, bo):
    tp = 512
    grid = _P1 // tp

    def body(x0r, x1r, x2r, w0r, w1r, w2r, br, gr, ber,
             whr, bhr, wor, bor, outr, zsum):
        y = (jnp.dot(x0r[...], w0r[...], preferred_element_type=jnp.float32)
             + jnp.dot(x1r[...], w1r[...], preferred_element_type=jnp.float32)
             + jnp.dot(x2r[...], w2r[...], preferred_element_type=jnp.float32)
             + br[...])
        y = _ln_relu(y, gr[...], ber[...])
        part = jnp.sum(y.reshape(tp, _B, 32), axis=0)

        @pl.when(pl.program_id(0) == 0)
        def _():
            zsum[...] = jnp.zeros_like(zsum)

        zsum[...] += part

        @pl.when(pl.program_id(0) == grid - 1)
        def _():
            z = zsum[...] * (1.0 / _P1)
            h = jnp.maximum(
                jnp.dot(z, whr[...], preferred_element_type=jnp.float32)
                + bhr[...], 0.0)
            outr[...] = (jnp.dot(h, wor[...],
                                 preferred_element_type=jnp.float32)
                         + bor[...])

    return pl.pallas_call(
        body,
        grid=(grid,),
        in_specs=[
            pl.BlockSpec((tp * _B, 32), lambda i: (i, 0)),
            pl.BlockSpec((tp * _B, 32), lambda i: (i, 0)),
            pl.BlockSpec((tp * _B, 32), lambda i: (i, 0)),
            pl.BlockSpec((32, 32), lambda i: (0, 0)),
            pl.BlockSpec((32, 32), lambda i: (0, 0)),
            pl.BlockSpec((32, 32), lambda i: (0, 0)),
            pl.BlockSpec((1, 32), lambda i: (0, 0)),
            pl.BlockSpec((1, 32), lambda i: (0, 0)),
            pl.BlockSpec((1, 32), lambda i: (0, 0)),
            pl.BlockSpec((32, 256), lambda i: (0, 0)),
            pl.BlockSpec((1, 256), lambda i: (0, 0)),
            pl.BlockSpec((256, 128), lambda i: (0, 0)),
            pl.BlockSpec((1, 128), lambda i: (0, 0)),
        ],
        out_specs=pl.BlockSpec((_B, 128), lambda i: (0, 0)),
        out_shape=jax.ShapeDtypeStruct((_B, 128), jnp.float32),
        scratch_shapes=[pltpu.VMEM((_B, 32), jnp.float32)],
    )(x0, x1, x2, w0, w1, w2, b, g, be, wh, bh, wo, bo)


def kernel(x, ei0, w0_lap, ei1, w1_lap, cW0, cb0, g0, be0,
           cW1, cb1, g1, be1, Wh, bh, Wo, bo):
    lap0 = _make_sc_lap(_P0, _B * 16)
    lap1 = _make_sc_lap(_P1, _B * 32)

    xt = jnp.transpose(x, (1, 0, 2)).reshape(_P0, _B * 16)
    src0 = ei0[0].reshape(-1, 128)
    src1 = ei1[0].reshape(-1, 128)

    x1t = lap0(xt, src0, w0_lap)
    y2t = lap0(x1t, src0, w0_lap)

    h0 = _combine0(xt.reshape(_P0 * _B, 16),
                   x1t.reshape(_P0 * _B, 16),
                   y2t.reshape(_P0 * _B, 16),
                   cW0[0] - cW0[2], cW0[1], 2.0 * cW0[2],
                   cb0.reshape(1, 32), g0.reshape(1, 32),
                   be0.reshape(1, 32))

    h0t = h0.reshape(_P1, _B * 32)
    h1t = lap1(h0t, src1, w1_lap)
    y2t1 = lap1(h1t, src1, w1_lap)

    return _combine1_head(
        h0, h1t.reshape(_P1 * _B, 32), y2t1.reshape(_P1 * _B, 32),
        cW1[0] - cW1[2], cW1[1], 2.0 * cW1[2],
        cb1.reshape(1, 32), g1.reshape(1, 32), be1.reshape(1, 32),
        Wh, bh.reshape(1, 256), Wo, bo.reshape(1, 128))


# speedup vs baseline: 374.4095x; 1.5775x over previous
import functools

import jax
import jax.numpy as jnp
from jax import lax
from jax.experimental import pallas as pl
from jax.experimental.pallas import tpu as pltpu
from jax.experimental.pallas import tpu_sc as plsc

_P0 = 49152
_P1 = 12288
_DEG = 8
_B = 4
_LANES = 16


def _make_sc_lap(P, D):
    ncores, nsub = 2, 16
    nw = ncores * nsub
    nodes_w = P // nw
    e_w = nodes_w * _DEG
    g_w = e_w // 128
    ec = 32768 // D
    cn = ec // _DEG
    nch = nodes_w // cn
    gpc = ec // 128
    nv = D // _LANES

    mesh = plsc.VectorSubcoreMesh(core_axis_name="c", subcore_axis_name="s",
                                  num_cores=ncores, num_subcores=nsub)
    out_type = jax.ShapeDtypeStruct((P, D), jnp.float32)
    scratch = [
        pltpu.VMEM((g_w, 128), jnp.int32),
        pltpu.VMEM((e_w + 8,), jnp.float32),
        pltpu.VMEM((2, ec, D), jnp.float32),
        pltpu.VMEM((cn, D), jnp.float32),
        pltpu.SemaphoreType.DMA,
        pltpu.SemaphoreType.DMA,
    ]

    def body(table, src, w, out, idx_v, w_v, rows_v, out_v, sem0, sem1):
        wid = lax.axis_index("s") * ncores + lax.axis_index("c")
        nbase = wid * nodes_w
        pltpu.sync_copy(src.at[pl.ds(wid * g_w, g_w)], idx_v)
        pltpu.sync_copy(w.at[pl.ds(wid * e_w, e_w)], w_v.at[pl.ds(0, e_w)])
        sems = [sem0, sem1]

        def issue(c, slot):
            descs = []
            for g in range(gpc):
                descs.append(pltpu.async_copy(
                    table.at[idx_v.at[c * gpc + g]],
                    rows_v.at[slot, pl.ds(g * 128, 128)],
                    sems[slot]))
            return descs

        pending = {0: issue(0, 0)}
        for c in range(nch):
            slot = c % 2
            if c + 1 < nch:
                pending[c + 1] = issue(c + 1, (c + 1) % 2)
            for dsc in pending.pop(c):
                dsc.wait()

            def node_body(n, carry, slot=slot, c=c):
                e0 = n * _DEG
                w16 = w_v[pl.ds(c * ec + e0, _LANES)]
                accs = [jnp.zeros((_LANES,), jnp.float32)
                        for _ in range(nv)]
                for d in range(_DEG):
                    wv = w16[d]
                    for j in range(nv):
                        row = rows_v[slot, e0 + d, pl.ds(j * _LANES, _LANES)]
                        accs[j] = accs[j] + row * wv
                for j in range(nv):
                    out_v[n, pl.ds(j * _LANES, _LANES)] = accs[j]
                return carry

            lax.fori_loop(0, cn, node_body, 0)
            pltpu.sync_copy(out_v, out.at[pl.ds(nbase + c * cn, cn)])

    return pl.kernel(body, out_type=out_type, mesh=mesh,
                     scratch_types=scratch,
                     compiler_params=pltpu.CompilerParams(
                         use_tc_tiling_on_sc=False))


def _ln_relu_lanes(y, s, g, be):
    m = jnp.dot(y, s, preferred_element_type=jnp.float32)
    q = jnp.dot(y * y, s, preferred_element_type=jnp.float32)
    v = q - m * m
    return jnp.maximum((y - m) * lax.rsqrt(v + 1e-5) * g + be, 0.0)


def _combine0(x0, x1, x2, k0, k1, k2, s, b, g, be):
    tp = 1024

    def body(x0r, x1r, x2r, k0r, k1r, k2r, sr, br, gr, ber, outr):
        y = (jnp.dot(x0r[...], k0r[...], preferred_element_type=jnp.float32)
             + jnp.dot(x1r[...], k1r[...], preferred_element_type=jnp.float32)
             + jnp.dot(x2r[...], k2r[...], preferred_element_type=jnp.float32)
             + br[...])
        y = _ln_relu_lanes(y, sr[...], gr[...], ber[...])
        ys = y.reshape(tp // 4, 4, 128)
        outr[...] = (ys[:, 0] + ys[:, 1] + ys[:, 2] + ys[:, 3]) * 0.25

    grid = _P0 // tp
    return pl.pallas_call(
        body,
        grid=(grid,),
        in_specs=[
            pl.BlockSpec((tp, 64), lambda i: (i, 0)),
            pl.BlockSpec((tp, 64), lambda i: (i, 0)),
            pl.BlockSpec((tp, 64), lambda i: (i, 0)),
            pl.BlockSpec((64, 128), lambda i: (0, 0)),
            pl.BlockSpec((64, 128), lambda i: (0, 0)),
            pl.BlockSpec((64, 128), lambda i: (0, 0)),
            pl.BlockSpec((128, 128), lambda i: (0, 0)),
            pl.BlockSpec((1, 128), lambda i: (0, 0)),
            pl.BlockSpec((1, 128), lambda i: (0, 0)),
            pl.BlockSpec((1, 128), lambda i: (0, 0)),
        ],
        out_specs=pl.BlockSpec((tp // 4, 128), lambda i: (i, 0)),
        out_shape=jax.ShapeDtypeStruct((_P1, 128), jnp.float32),
    )(x0, x1, x2, k0, k1, k2, s, b, g, be)


def _combine1_head(x0, x1, x2, k0, k1, k2, s, b, g, be, wh, bh, wo, bo):
    tp = 1024
    grid = _P1 // tp

    def body(x0r, x1r, x2r, k0r, k1r, k2r, sr, br, gr, ber,
             whr, bhr, wor, bor, outr, zsum):
        y = (jnp.dot(x0r[...], k0r[...], preferred_element_type=jnp.float32)
             + jnp.dot(x1r[...], k1r[...], preferred_element_type=jnp.float32)
             + jnp.dot(x2r[...], k2r[...], preferred_element_type=jnp.float32)
             + br[...])
        y = _ln_relu_lanes(y, sr[...], gr[...], ber[...])
        part = jnp.sum(y, axis=0, keepdims=True)

        @pl.when(pl.program_id(0) == 0)
        def _():
            zsum[...] = jnp.zeros_like(zsum)

        zsum[...] += part

        @pl.when(pl.program_id(0) == grid - 1)
        def _():
            z = zsum[...] * (1.0 / _P1)
            h = jnp.maximum(
                jnp.dot(z, whr[...], preferred_element_type=jnp.float32)
                + bhr[...], 0.0)
            outr[...] = (jnp.dot(h, wor[...],
                                 preferred_element_type=jnp.float32)
                         + bor[...])

    return pl.pallas_call(
        body,
        grid=(grid,),
        in_specs=[
            pl.BlockSpec((tp, 128), lambda i: (i, 0)),
            pl.BlockSpec((tp, 128), lambda i: (i, 0)),
            pl.BlockSpec((tp, 128), lambda i: (i, 0)),
            pl.BlockSpec((128, 128), lambda i: (0, 0)),
            pl.BlockSpec((128, 128), lambda i: (0, 0)),
            pl.BlockSpec((128, 128), lambda i: (0, 0)),
            pl.BlockSpec((128, 128), lambda i: (0, 0)),
            pl.BlockSpec((1, 128), lambda i: (0, 0)),
            pl.BlockSpec((1, 128), lambda i: (0, 0)),
            pl.BlockSpec((1, 128), lambda i: (0, 0)),
            pl.BlockSpec((128, _B * 256), lambda i: (0, 0)),
            pl.BlockSpec((1, _B * 256), lambda i: (0, 0)),
            pl.BlockSpec((_B * 256, _B * 128), lambda i: (0, 0)),
            pl.BlockSpec((1, _B * 128), lambda i: (0, 0)),
        ],
        out_specs=pl.BlockSpec((1, _B * 128), lambda i: (0, 0)),
        out_shape=jax.ShapeDtypeStruct((1, _B * 128), jnp.float32),
        scratch_shapes=[pltpu.VMEM((1, 128), jnp.float32)],
    )(x0, x1, x2, k0, k1, k2, s, b, g, be, wh, bh, wo, bo)


def kernel(x, ei0, w0_lap, ei1, w1_lap, cW0, cb0, g0, be0,
           cW1, cb1, g1, be1, Wh, bh, Wo, bo):
    lap0 = _make_sc_lap(_P0, _B * 16)
    lap1 = _make_sc_lap(_P1, _B * 32)

    xt = jnp.transpose(x, (1, 0, 2)).reshape(_P0, _B * 16)
    src0 = ei0[0].reshape(-1, 128)
    src1 = ei1[0].reshape(-1, 128)

    x1t = lap0(xt, src0, w0_lap)
    y2t = lap0(x1t, src0, w0_lap)

    eye = jnp.eye(_B, dtype=jnp.float32)
    seg32 = jnp.kron(eye, jnp.full((32, 32), 1.0 / 32, jnp.float32))

    h0t = _combine0(
        xt, x1t, y2t,
        jnp.kron(eye, cW0[0] - cW0[2]), jnp.kron(eye, cW0[1]),
        jnp.kron(eye, 2.0 * cW0[2]), seg32,
        jnp.tile(cb0, _B).reshape(1, -1), jnp.tile(g0, _B).reshape(1, -1),
        jnp.tile(be0, _B).reshape(1, -1))

    h1t = lap1(h0t, src1, w1_lap)
    y2t1 = lap1(h1t, src1, w1_lap)

    out = _combine1_head(
        h0t, h1t, y2t1,
        jnp.kron(eye, cW1[0] - cW1[2]), jnp.kron(eye, cW1[1]),
        jnp.kron(eye, 2.0 * cW1[2]), seg32,
        jnp.tile(cb1, _B).reshape(1, -1), jnp.tile(g1, _B).reshape(1, -1),
        jnp.tile(be1, _B).reshape(1, -1),
        jnp.kron(eye, Wh), jnp.tile(bh, _B).reshape(1, -1),
        jnp.kron(eye, Wo), jnp.tile(bo, _B).reshape(1, -1))
    return out.reshape(_B, 128)
